# Initial kernel scaffold; baseline (speedup 1.0000x reference)
import jax
import jax.numpy as jnp
from jax.experimental import pallas as pl


def kernel(inputs, embeddings):
    raise NotImplementedError("write your pallas kernel here")


# speedup vs baseline: 1.0235x; 1.0235x over previous
import functools

import jax
import jax.numpy as jnp
from jax import lax
from jax.experimental import pallas as pl
from jax.experimental.pallas import tpu as pltpu
from jax.experimental.pallas import tpu_sc as plsc

D = 32
CHUNK = 128
NW = 32


@functools.partial(jax.jit, static_argnames=("total",))
def _sc_gather(idx, table, total):
    per_w = total // NW
    nchunks = per_w // CHUNK
    mesh = plsc.VectorSubcoreMesh(core_axis_name="c", subcore_axis_name="s")

    @functools.partial(
        pl.kernel,
        mesh=mesh,
        out_type=jax.ShapeDtypeStruct((total, D), jnp.float32),
        scratch_types=[
            pltpu.VMEM((nchunks, CHUNK), jnp.int32),
            pltpu.VMEM((CHUNK, D), jnp.float32),
            pltpu.SemaphoreType.DMA,
        ],
        compiler_params=pltpu.CompilerParams(use_tc_tiling_on_sc=False),
    )
    def k(idx_hbm, table_hbm, out_hbm, idx_v, rows_v, sem):
        wid = lax.axis_index("s") * 2 + lax.axis_index("c")
        base = wid * per_w
        pltpu.sync_copy(idx_hbm.at[wid], idx_v)

        def body(j, carry):
            pltpu.async_copy(table_hbm.at[idx_v.at[j]], rows_v, sem).wait()
            pltpu.sync_copy(rows_v, out_hbm.at[pl.ds(base + j * CHUNK, CHUNK)])
            return carry

        lax.fori_loop(0, nchunks, body, 0)

    return k(idx, table)


def kernel(inputs, embeddings):
    batch, hist = inputs.shape
    total = batch * hist
    idx = inputs.astype(jnp.int32).reshape(NW, (total // NW) // CHUNK, CHUNK)
    out = _sc_gather(idx, embeddings, total)
    return out.reshape(batch, hist, D)


# speedup vs baseline: 1.1144x; 1.0888x over previous
import functools

import jax
import jax.numpy as jnp
from jax import lax
from jax.experimental import pallas as pl
from jax.experimental.pallas import tpu as pltpu
from jax.experimental.pallas import tpu_sc as plsc

D = 32
CHUNK = 128
NBUF = 8
NW = 32


@functools.partial(jax.jit, static_argnames=("total",))
def _sc_gather(idx, table, total):
    per_w = total // NW
    nchunks = per_w // CHUNK
    ngroups = nchunks // NBUF
    mesh = plsc.VectorSubcoreMesh(core_axis_name="c", subcore_axis_name="s")

    @functools.partial(
        pl.kernel,
        mesh=mesh,
        out_type=jax.ShapeDtypeStruct((total, D), jnp.float32),
        scratch_types=[
            pltpu.VMEM((nchunks, CHUNK), jnp.int32),
            pltpu.VMEM((NBUF, CHUNK, D), jnp.float32),
            [pltpu.SemaphoreType.DMA] * NBUF,
            [pltpu.SemaphoreType.DMA] * NBUF,
        ],
        compiler_params=pltpu.CompilerParams(use_tc_tiling_on_sc=False),
    )
    def k(idx_hbm, table_hbm, out_hbm, idx_v, rows_v, gsems, wsems):
        wid = lax.axis_index("s") * 2 + lax.axis_index("c")
        base = wid * per_w
        pltpu.sync_copy(idx_hbm.at[wid], idx_v)

        def start_gather(j, b):
            pltpu.async_copy(table_hbm.at[idx_v.at[j]], rows_v.at[b], gsems[b])

        def wait_gather(b):
            pltpu.make_async_copy(
                table_hbm.at[pl.ds(0, CHUNK)], rows_v.at[b], gsems[b]
            ).wait()

        def start_write(j, b):
            pltpu.async_copy(
                rows_v.at[b], out_hbm.at[pl.ds(base + j * CHUNK, CHUNK)], wsems[b]
            )

        def wait_write(b):
            pltpu.make_async_copy(
                rows_v.at[b], out_hbm.at[pl.ds(base, CHUNK)], wsems[b]
            ).wait()

        for b in range(NBUF):
            start_gather(b, b)

        def body(g, carry):
            for b in range(NBUF):
                j = g * NBUF + b
                wait_gather(b)
                start_write(j, b)
                nxt = j + NBUF

                @pl.when(nxt < nchunks)
                def _():
                    wait_write(b)
                    start_gather(nxt, b)

            return carry

        lax.fori_loop(0, ngroups, body, 0)
        for b in range(NBUF):
            wait_write(b)

    return k(idx, table)


def kernel(inputs, embeddings):
    batch, hist = inputs.shape
    total = batch * hist
    idx = inputs.astype(jnp.int32).reshape(NW, (total // NW) // CHUNK, CHUNK)
    out = _sc_gather(idx, embeddings, total)
    return out.reshape(batch, hist, D)


# speedup vs baseline: 1.7922x; 1.6082x over previous
import functools

import jax
import jax.numpy as jnp
from jax import lax
from jax.experimental import pallas as pl
from jax.experimental.pallas import tpu as pltpu
from jax.experimental.pallas import tpu_sc as plsc

D = 32
NBUF = 8
NW = 32


@functools.partial(jax.jit, static_argnames=("batch", "hist"))
def _sc_gather(idx, table, batch, hist):
    rows_per_w = batch // NW
    ngroups = rows_per_w // NBUF
    mesh = plsc.VectorSubcoreMesh(core_axis_name="c", subcore_axis_name="s")

    @functools.partial(
        pl.kernel,
        mesh=mesh,
        out_type=jax.ShapeDtypeStruct((batch, hist, D), jnp.float32),
        scratch_types=[
            pltpu.VMEM((rows_per_w, hist), jnp.int32),
            pltpu.VMEM((NBUF, hist, D), jnp.float32),
            [pltpu.SemaphoreType.DMA] * NBUF,
            [pltpu.SemaphoreType.DMA] * NBUF,
        ],
        compiler_params=pltpu.CompilerParams(use_tc_tiling_on_sc=False),
    )
    def k(idx_hbm, table_hbm, out_hbm, idx_v, rows_v, gsems, wsems):
        wid = lax.axis_index("s") * 2 + lax.axis_index("c")
        base = wid * rows_per_w
        pltpu.sync_copy(idx_hbm.at[pl.ds(base, rows_per_w)], idx_v)

        def start_gather(r, b):
            pltpu.async_copy(table_hbm.at[idx_v.at[r]], rows_v.at[b], gsems[b])

        def wait_gather(b):
            pltpu.make_async_copy(
                table_hbm.at[pl.ds(0, hist)], rows_v.at[b], gsems[b]
            ).wait()

        def start_write(r, b):
            pltpu.async_copy(rows_v.at[b], out_hbm.at[base + r], wsems[b])

        def wait_write(b):
            pltpu.make_async_copy(rows_v.at[b], out_hbm.at[base], wsems[b]).wait()

        for b in range(NBUF):
            start_gather(b, b)

        def body(g, carry):
            for b in range(NBUF):
                r = g * NBUF + b
                wait_gather(b)
                start_write(r, b)
                nxt = r + NBUF

                @pl.when(nxt < rows_per_w)
                def _():
                    wait_write(b)
                    start_gather(nxt, b)

            return carry

        lax.fori_loop(0, ngroups, body, 0)
        for b in range(NBUF):
            wait_write(b)

    return k(idx, table)


def kernel(inputs, embeddings):
    batch, hist = inputs.shape
    return _sc_gather(inputs.astype(jnp.int32), embeddings, batch, hist)
